# baseline (speedup 1.0000x reference)
import functools

import jax
import jax.numpy as jnp
from jax import lax
from jax.experimental import pallas as pl
from jax.experimental.pallas import tpu as pltpu
from jax.experimental.pallas import tpu_sc as plsc

N_NODES = 10000
N_EDGES = 320000
IN_FEATS = 128
N_HIDDEN = 128
OUT_FEATS = 64
F = 64

NC = 2
NS = 16
NW = NC * NS
EW = N_EDGES // NW
K = 125
NCHUNK = EW // K
N_PAD = 10240
ROWS_PER_SUB = N_PAD // NS
ZROWS = 128

_MESH = dict(core_axis_name="c", subcore_axis_name="s", num_cores=NC,
             num_subcores=NS)
_SC_PARAMS = pltpu.CompilerParams(needs_layout_passes=False,
                                  use_tc_tiling_on_sc=False)


def _degrees(src, dst):
    mesh = plsc.VectorSubcoreMesh(**_MESH)

    @functools.partial(
        pl.kernel,
        out_type=[jax.ShapeDtypeStruct((NW, N_PAD), jnp.float32),
                  jax.ShapeDtypeStruct((NW, N_PAD), jnp.float32)],
        mesh=mesh,
        scratch_types=[pltpu.VMEM((EW,), jnp.int32),
                       pltpu.VMEM((N_PAD,), jnp.float32)],
        compiler_params=_SC_PARAMS,
    )
    def k(src_hbm, dst_hbm, out_o, out_i, idx_v, hist_v):
        c = lax.axis_index("c")
        s = lax.axis_index("s")
        wid = s * NC + c
        base = wid * EW
        zeros16 = jnp.zeros((16,), jnp.float32)
        ones16 = jnp.full((16,), 1.0, jnp.float32)

        for in_hbm, out_hbm in ((src_hbm, out_o), (dst_hbm, out_i)):
            def zero_body(i, _):
                hist_v[pl.ds(i * 16, 16)] = zeros16
                return 0

            lax.fori_loop(0, N_PAD // 16, zero_body, 0)
            pltpu.sync_copy(in_hbm.at[pl.ds(base, EW)], idx_v)

            def add_body(i, _):
                idx = idx_v[pl.ds(i * 16, 16)]
                plsc.addupdate_scatter(hist_v, [idx], ones16)
                return 0

            lax.fori_loop(0, EW // 16, add_body, 0)
            pltpu.sync_copy(hist_v, out_hbm.at[wid])

    return k(src, dst)


_SPMM_SCRATCH = [pltpu.VMEM((NCHUNK, K), jnp.int32),
                 pltpu.VMEM((NCHUNK, K), jnp.int32),
                 [pltpu.VMEM((K, F), jnp.float32)] * 4,
                 pltpu.VMEM((ZROWS, F), jnp.float32),
                 pltpu.VMEM_SHARED((N_PAD, F), jnp.float32),
                 [pltpu.SemaphoreType.DMA] * 4,
                 [pltpu.SemaphoreType.DMA] * 4,
                 pltpu.SemaphoreType.DMA,
                 pltpu.SemaphoreType.DMA]


def _fill_zbuf(zbuf):
    zeros16 = jnp.zeros((16,), jnp.float32)

    def zrow(r, _):
        def zcol(j, _):
            zbuf[r, pl.ds(j * 16, 16)] = zeros16
            return 0
        lax.fori_loop(0, F // 16, zcol, 0)
        return 0

    lax.fori_loop(0, ZROWS, zrow, 0)


def _zero_acc_slice(zbuf, acc, s):
    for t in range(ROWS_PER_SUB // ZROWS):
        pltpu.sync_copy(
            zbuf, acc.at[pl.ds(s * ROWS_PER_SUB + t * ZROWS, ZROWS)])


def _dump_acc(acc, out_hbm, c, s):
    pltpu.sync_copy(acc.at[pl.ds(s * ROWS_PER_SUB, ROWS_PER_SUB)],
                    out_hbm.at[c, pl.ds(s * ROWS_PER_SUB, ROWS_PER_SUB)])


def _ring(hs_hbm, src_v, dst_v, rows, acc, gsem, ssem):
    def fire_gather(j, b):
        pltpu.async_copy(hs_hbm.at[src_v.at[j]], rows[b], gsem[b])

    def wait_gather(j, b):
        pltpu.make_async_copy(hs_hbm.at[src_v.at[j]], rows[b],
                              gsem[b]).wait()

    def fire_scatter(j, b):
        pltpu.async_copy(rows[b], acc.at[dst_v.at[j]], ssem[b], add=True)

    def wait_scatter(j, b):
        pltpu.make_async_copy(rows[b], acc.at[dst_v.at[j]], ssem[b]).wait()

    def step(j, b, *, do_wait_s=True, do_fire_g=True):
        b2 = (b + 2) % 4
        if do_wait_s:
            wait_scatter(j - 2, b2)
        if do_fire_g:
            fire_gather(j + 2, b2)
        wait_gather(j, b)
        fire_scatter(j, b)

    fire_gather(0, 0)
    fire_gather(1, 1)
    step(0, 0, do_wait_s=False)
    step(1, 1, do_wait_s=False)
    step(2, 2)
    step(3, 3)

    def body(i, _):
        j = 4 * i
        step(j + 0, 0)
        step(j + 1, 1)
        step(j + 2, 2)
        step(j + 3, 3)
        return 0

    lax.fori_loop(1, NCHUNK // 4 - 1, body, 0)

    jl = NCHUNK - 4
    step(jl + 0, 0)
    step(jl + 1, 1)
    step(jl + 2, 2, do_fire_g=False)
    step(jl + 3, 3, do_fire_g=False)
    wait_scatter(NCHUNK - 2, 2)
    wait_scatter(NCHUNK - 1, 3)


_SPMM_KERNELS = {}


def _spmm_layer1(hs_a, hs_b, srcs, dsts):
    if 1 not in _SPMM_KERNELS:
        mesh = plsc.VectorSubcoreMesh(**_MESH)

        @functools.partial(
            pl.kernel,
            out_type=[jax.ShapeDtypeStruct((NC, N_PAD, F), jnp.float32),
                      jax.ShapeDtypeStruct((NC, N_PAD, F), jnp.float32)],
            mesh=mesh,
            scratch_types=_SPMM_SCRATCH,
            compiler_params=_SC_PARAMS,
        )
        def k(hsa_hbm, hsb_hbm, src_hbm, dst_hbm, outa_hbm, outb_hbm,
              src_v, dst_v, rows, zbuf, acc, gsem, ssem, isem_s, isem_d):
            c = lax.axis_index("c")
            s = lax.axis_index("s")
            wid = s * NC + c

            pltpu.async_copy(src_hbm.at[wid], src_v, isem_s)
            pltpu.async_copy(dst_hbm.at[wid], dst_v, isem_d)
            _fill_zbuf(zbuf)
            _zero_acc_slice(zbuf, acc, s)
            pltpu.make_async_copy(src_hbm.at[wid], src_v, isem_s).wait()
            pltpu.make_async_copy(dst_hbm.at[wid], dst_v, isem_d).wait()
            plsc.subcore_barrier()

            _ring(hsa_hbm, src_v, dst_v, rows, acc, gsem, ssem)
            plsc.subcore_barrier()
            _dump_acc(acc, outa_hbm, c, s)
            _zero_acc_slice(zbuf, acc, s)
            plsc.subcore_barrier()

            _ring(hsb_hbm, src_v, dst_v, rows, acc, gsem, ssem)
            plsc.subcore_barrier()
            _dump_acc(acc, outb_hbm, c, s)

        _SPMM_KERNELS[1] = k
    return _SPMM_KERNELS[1](hs_a, hs_b, srcs, dsts)


def _spmm_layer2(hs, srcs, dsts):
    if 2 not in _SPMM_KERNELS:
        mesh = plsc.VectorSubcoreMesh(**_MESH)

        @functools.partial(
            pl.kernel,
            out_type=jax.ShapeDtypeStruct((NC, N_PAD, F), jnp.float32),
            mesh=mesh,
            scratch_types=_SPMM_SCRATCH,
            compiler_params=_SC_PARAMS,
        )
        def k(hs_hbm, src_hbm, dst_hbm, out_hbm,
              src_v, dst_v, rows, zbuf, acc, gsem, ssem, isem_s, isem_d):
            c = lax.axis_index("c")
            s = lax.axis_index("s")
            wid = s * NC + c

            pltpu.async_copy(src_hbm.at[wid], src_v, isem_s)
            pltpu.async_copy(dst_hbm.at[wid], dst_v, isem_d)
            _fill_zbuf(zbuf)
            _zero_acc_slice(zbuf, acc, s)
            pltpu.make_async_copy(src_hbm.at[wid], src_v, isem_s).wait()
            pltpu.make_async_copy(dst_hbm.at[wid], dst_v, isem_d).wait()
            plsc.subcore_barrier()

            _ring(hs_hbm, src_v, dst_v, rows, acc, gsem, ssem)
            plsc.subcore_barrier()
            _dump_acc(acc, out_hbm, c, s)

        _SPMM_KERNELS[2] = k
    return _SPMM_KERNELS[2](hs, srcs, dsts)


_R = 1024
_GRID = (N_PAD // _R,)


def _rsqrt_deg(parts_ref):
    i = pl.program_id(0)
    deg = jnp.sum(parts_ref[:, pl.ds(i * _R, _R)], axis=0)
    return lax.rsqrt(jnp.maximum(deg, 1.0))


def _hist_spec():
    return pl.BlockSpec((NW, N_PAD), lambda i: (0, 0))


def _mm_scale_body(x_ref, w_ref, ho_ref, out_a_ref, out_b_ref):
    r_out = _rsqrt_deg(ho_ref)
    h = jnp.dot(x_ref[...], w_ref[...], preferred_element_type=jnp.float32)
    hs = h * r_out[:, None]
    out_a_ref[...] = hs[:, :F]
    out_b_ref[...] = hs[:, F:]


def _first_stage(x, W0, ho_parts):
    return pl.pallas_call(
        _mm_scale_body,
        grid=_GRID,
        in_specs=[pl.BlockSpec((_R, IN_FEATS), lambda i: (i, 0)),
                  pl.BlockSpec((IN_FEATS, N_HIDDEN), lambda i: (0, 0)),
                  _hist_spec()],
        out_specs=[pl.BlockSpec((_R, F), lambda i: (i, 0)),
                   pl.BlockSpec((_R, F), lambda i: (i, 0))],
        out_shape=[jax.ShapeDtypeStruct((N_NODES, F), jnp.float32),
                   jax.ShapeDtypeStruct((N_NODES, F), jnp.float32)],
    )(x, W0, ho_parts)


def _mid_body(pa_ref, pb_ref, hi_ref, ho_ref, b0_ref, w1_ref, out_ref):
    r_in = _rsqrt_deg(hi_ref)
    r_out = _rsqrt_deg(ho_ref)
    ssum = jnp.concatenate([pa_ref[0] + pa_ref[1], pb_ref[0] + pb_ref[1]],
                           axis=1)
    h1 = jnp.maximum(ssum * r_in[:, None] + b0_ref[...][None, :], 0.0)
    h = jnp.dot(h1, w1_ref[...], preferred_element_type=jnp.float32)
    out_ref[...] = h * r_out[:, None]


def _mid_stage(parts_a, parts_b, hi_parts, ho_parts, b0, W1):
    return pl.pallas_call(
        _mid_body,
        grid=_GRID,
        in_specs=[pl.BlockSpec((NC, _R, F), lambda i: (0, i, 0)),
                  pl.BlockSpec((NC, _R, F), lambda i: (0, i, 0)),
                  _hist_spec(),
                  _hist_spec(),
                  pl.BlockSpec((N_HIDDEN,), lambda i: (0,)),
                  pl.BlockSpec((N_HIDDEN, OUT_FEATS), lambda i: (0, 0))],
        out_specs=pl.BlockSpec((_R, OUT_FEATS), lambda i: (i, 0)),
        out_shape=jax.ShapeDtypeStruct((N_NODES, OUT_FEATS), jnp.float32),
    )(parts_a, parts_b, hi_parts, ho_parts, b0, W1)


def _final_body(parts_ref, hi_ref, b1_ref, out_ref):
    r_in = _rsqrt_deg(hi_ref)
    t = ((parts_ref[0] + parts_ref[1]) * r_in[:, None]
         + b1_ref[...][None, :])
    m = jnp.max(t, axis=1, keepdims=True)
    e = jnp.exp(t - m)
    out_ref[...] = (t - m) - jnp.log(jnp.sum(e, axis=1, keepdims=True))


def _final_stage(parts2, hi_parts, b1):
    return pl.pallas_call(
        _final_body,
        grid=_GRID,
        in_specs=[pl.BlockSpec((NC, _R, OUT_FEATS), lambda i: (0, i, 0)),
                  _hist_spec(),
                  pl.BlockSpec((OUT_FEATS,), lambda i: (0,))],
        out_specs=pl.BlockSpec((_R, OUT_FEATS), lambda i: (i, 0)),
        out_shape=jax.ShapeDtypeStruct((N_NODES, OUT_FEATS), jnp.float32),
    )(parts2, hi_parts, b1)


def kernel(input_feature, edge_index, W0, b0, W1, b1):
    src = edge_index[0]
    dst = edge_index[1]
    srcs = src.reshape(NW, NCHUNK, K)
    dsts = dst.reshape(NW, NCHUNK, K)

    ho_parts, hi_parts = _degrees(src, dst)

    hs0_a, hs0_b = _first_stage(input_feature, W0, ho_parts)
    parts_a, parts_b = _spmm_layer1(hs0_a, hs0_b, srcs, dsts)
    hs1 = _mid_stage(parts_a, parts_b, hi_parts, ho_parts, b0, W1)
    parts2 = _spmm_layer2(hs1, srcs, dsts)
    return _final_stage(parts2, hi_parts, b1)


# speedup vs baseline: 1.0223x; 1.0223x over previous
import functools

import jax
import jax.numpy as jnp
from jax import lax
from jax.experimental import pallas as pl
from jax.experimental.pallas import tpu as pltpu
from jax.experimental.pallas import tpu_sc as plsc

N_NODES = 10000
N_EDGES = 320000
IN_FEATS = 128
N_HIDDEN = 128
OUT_FEATS = 64
F = 64

NC = 2
NS = 16
NW = NC * NS
EW = N_EDGES // NW
K = 125
NCHUNK = EW // K
NBUF = 8
G = 4
N_PAD = 10240
ROWS_PER_SUB = N_PAD // NS
ZROWS = 64

_MESH = dict(core_axis_name="c", subcore_axis_name="s", num_cores=NC,
             num_subcores=NS)
_SC_PARAMS = pltpu.CompilerParams(needs_layout_passes=False,
                                  use_tc_tiling_on_sc=False)


def _degrees(src, dst):
    mesh = plsc.VectorSubcoreMesh(**_MESH)

    @functools.partial(
        pl.kernel,
        out_type=[jax.ShapeDtypeStruct((NW, N_PAD), jnp.float32),
                  jax.ShapeDtypeStruct((NW, N_PAD), jnp.float32)],
        mesh=mesh,
        scratch_types=[pltpu.VMEM((EW,), jnp.int32),
                       pltpu.VMEM((N_PAD,), jnp.float32)],
        compiler_params=_SC_PARAMS,
    )
    def k(src_hbm, dst_hbm, out_o, out_i, idx_v, hist_v):
        c = lax.axis_index("c")
        s = lax.axis_index("s")
        wid = s * NC + c
        base = wid * EW
        zeros16 = jnp.zeros((16,), jnp.float32)
        ones16 = jnp.full((16,), 1.0, jnp.float32)

        for in_hbm, out_hbm in ((src_hbm, out_o), (dst_hbm, out_i)):
            def zero_body(i, _):
                hist_v[pl.ds(i * 16, 16)] = zeros16
                return 0

            lax.fori_loop(0, N_PAD // 16, zero_body, 0)
            pltpu.sync_copy(in_hbm.at[pl.ds(base, EW)], idx_v)

            def add_body(i, _):
                idx = idx_v[pl.ds(i * 16, 16)]
                plsc.addupdate_scatter(hist_v, [idx], ones16)
                return 0

            lax.fori_loop(0, EW // 16, add_body, 0)
            pltpu.sync_copy(hist_v, out_hbm.at[wid])

    return k(src, dst)


_SPMM_SCRATCH = [pltpu.VMEM((NCHUNK, K), jnp.int32),
                 pltpu.VMEM((NCHUNK, K), jnp.int32),
                 [pltpu.VMEM((K, F), jnp.float32)] * NBUF,
                 pltpu.VMEM((ZROWS, F), jnp.float32),
                 pltpu.VMEM_SHARED((N_PAD, F), jnp.float32),
                 [pltpu.SemaphoreType.DMA] * NBUF,
                 [pltpu.SemaphoreType.DMA] * NBUF,
                 pltpu.SemaphoreType.DMA,
                 pltpu.SemaphoreType.DMA]


def _fill_zbuf(zbuf):
    zeros16 = jnp.zeros((16,), jnp.float32)

    def zrow(r, _):
        def zcol(j, _):
            zbuf[r, pl.ds(j * 16, 16)] = zeros16
            return 0
        lax.fori_loop(0, F // 16, zcol, 0)
        return 0

    lax.fori_loop(0, ZROWS, zrow, 0)


def _zero_acc_slice(zbuf, acc, s):
    for t in range(ROWS_PER_SUB // ZROWS):
        pltpu.sync_copy(
            zbuf, acc.at[pl.ds(s * ROWS_PER_SUB + t * ZROWS, ZROWS)])


def _dump_acc(acc, out_hbm, c, s):
    pltpu.sync_copy(acc.at[pl.ds(s * ROWS_PER_SUB, ROWS_PER_SUB)],
                    out_hbm.at[c, pl.ds(s * ROWS_PER_SUB, ROWS_PER_SUB)])


def _ring(hs_hbm, src_v, dst_v, rows, acc, gsem, ssem):
    def fire_gather(j, b):
        pltpu.async_copy(hs_hbm.at[src_v.at[j]], rows[b], gsem[b])

    def wait_gather(j, b):
        pltpu.make_async_copy(hs_hbm.at[src_v.at[j]], rows[b],
                              gsem[b]).wait()

    def fire_scatter(j, b):
        pltpu.async_copy(rows[b], acc.at[dst_v.at[j]], ssem[b], add=True)

    def wait_scatter(j, b):
        pltpu.make_async_copy(rows[b], acc.at[dst_v.at[j]], ssem[b]).wait()

    def step(j, b, *, do_wait_s=True, do_fire_g=True):
        bg = (b + G) % NBUF
        if do_wait_s:
            wait_scatter(j + G - NBUF, bg)
        if do_fire_g:
            fire_gather(j + G, bg)
        wait_gather(j, b)
        fire_scatter(j, b)

    for j in range(G):
        fire_gather(j, j)
    for b in range(NBUF):
        step(b, b, do_wait_s=(b + G - NBUF >= 0))

    def body(i, _):
        j0 = NBUF * i
        for b in range(NBUF):
            step(j0 + b, b)
        return 0

    lax.fori_loop(1, NCHUNK // NBUF - 1, body, 0)

    jl = NCHUNK - NBUF
    for b in range(NBUF):
        step(jl + b, b, do_fire_g=(jl + b + G < NCHUNK))
    for j in range(NCHUNK - (NBUF - G), NCHUNK):
        wait_scatter(j, j % NBUF)


_SPMM_KERNELS = {}


def _spmm_layer1(hs_a, hs_b, srcs, dsts):
    if 1 not in _SPMM_KERNELS:
        mesh = plsc.VectorSubcoreMesh(**_MESH)

        @functools.partial(
            pl.kernel,
            out_type=[jax.ShapeDtypeStruct((NC, N_PAD, F), jnp.float32),
                      jax.ShapeDtypeStruct((NC, N_PAD, F), jnp.float32)],
            mesh=mesh,
            scratch_types=_SPMM_SCRATCH,
            compiler_params=_SC_PARAMS,
        )
        def k(hsa_hbm, hsb_hbm, src_hbm, dst_hbm, outa_hbm, outb_hbm,
              src_v, dst_v, rows, zbuf, acc, gsem, ssem, isem_s, isem_d):
            c = lax.axis_index("c")
            s = lax.axis_index("s")
            wid = s * NC + c

            pltpu.async_copy(src_hbm.at[wid], src_v, isem_s)
            pltpu.async_copy(dst_hbm.at[wid], dst_v, isem_d)
            _fill_zbuf(zbuf)
            _zero_acc_slice(zbuf, acc, s)
            pltpu.make_async_copy(src_hbm.at[wid], src_v, isem_s).wait()
            pltpu.make_async_copy(dst_hbm.at[wid], dst_v, isem_d).wait()
            plsc.subcore_barrier()

            _ring(hsa_hbm, src_v, dst_v, rows, acc, gsem, ssem)
            plsc.subcore_barrier()
            _dump_acc(acc, outa_hbm, c, s)
            _zero_acc_slice(zbuf, acc, s)
            plsc.subcore_barrier()

            _ring(hsb_hbm, src_v, dst_v, rows, acc, gsem, ssem)
            plsc.subcore_barrier()
            _dump_acc(acc, outb_hbm, c, s)

        _SPMM_KERNELS[1] = k
    return _SPMM_KERNELS[1](hs_a, hs_b, srcs, dsts)


def _spmm_layer2(hs, srcs, dsts):
    if 2 not in _SPMM_KERNELS:
        mesh = plsc.VectorSubcoreMesh(**_MESH)

        @functools.partial(
            pl.kernel,
            out_type=jax.ShapeDtypeStruct((NC, N_PAD, F), jnp.float32),
            mesh=mesh,
            scratch_types=_SPMM_SCRATCH,
            compiler_params=_SC_PARAMS,
        )
        def k(hs_hbm, src_hbm, dst_hbm, out_hbm,
              src_v, dst_v, rows, zbuf, acc, gsem, ssem, isem_s, isem_d):
            c = lax.axis_index("c")
            s = lax.axis_index("s")
            wid = s * NC + c

            pltpu.async_copy(src_hbm.at[wid], src_v, isem_s)
            pltpu.async_copy(dst_hbm.at[wid], dst_v, isem_d)
            _fill_zbuf(zbuf)
            _zero_acc_slice(zbuf, acc, s)
            pltpu.make_async_copy(src_hbm.at[wid], src_v, isem_s).wait()
            pltpu.make_async_copy(dst_hbm.at[wid], dst_v, isem_d).wait()
            plsc.subcore_barrier()

            _ring(hs_hbm, src_v, dst_v, rows, acc, gsem, ssem)
            plsc.subcore_barrier()
            _dump_acc(acc, out_hbm, c, s)

        _SPMM_KERNELS[2] = k
    return _SPMM_KERNELS[2](hs, srcs, dsts)


_R = 1024
_GRID = (N_PAD // _R,)


def _rsqrt_deg(parts_ref):
    i = pl.program_id(0)
    deg = jnp.sum(parts_ref[:, pl.ds(i * _R, _R)], axis=0)
    return lax.rsqrt(jnp.maximum(deg, 1.0))


def _hist_spec():
    return pl.BlockSpec((NW, N_PAD), lambda i: (0, 0))


def _mm_scale_body(x_ref, w_ref, ho_ref, out_a_ref, out_b_ref):
    r_out = _rsqrt_deg(ho_ref)
    h = jnp.dot(x_ref[...], w_ref[...], preferred_element_type=jnp.float32)
    hs = h * r_out[:, None]
    out_a_ref[...] = hs[:, :F]
    out_b_ref[...] = hs[:, F:]


def _first_stage(x, W0, ho_parts):
    return pl.pallas_call(
        _mm_scale_body,
        grid=_GRID,
        in_specs=[pl.BlockSpec((_R, IN_FEATS), lambda i: (i, 0)),
                  pl.BlockSpec((IN_FEATS, N_HIDDEN), lambda i: (0, 0)),
                  _hist_spec()],
        out_specs=[pl.BlockSpec((_R, F), lambda i: (i, 0)),
                   pl.BlockSpec((_R, F), lambda i: (i, 0))],
        out_shape=[jax.ShapeDtypeStruct((N_NODES, F), jnp.float32),
                   jax.ShapeDtypeStruct((N_NODES, F), jnp.float32)],
    )(x, W0, ho_parts)


def _mid_body(pa_ref, pb_ref, hi_ref, ho_ref, b0_ref, w1_ref, out_ref):
    r_in = _rsqrt_deg(hi_ref)
    r_out = _rsqrt_deg(ho_ref)
    ssum = jnp.concatenate([pa_ref[0] + pa_ref[1], pb_ref[0] + pb_ref[1]],
                           axis=1)
    h1 = jnp.maximum(ssum * r_in[:, None] + b0_ref[...][None, :], 0.0)
    h = jnp.dot(h1, w1_ref[...], preferred_element_type=jnp.float32)
    out_ref[...] = h * r_out[:, None]


def _mid_stage(parts_a, parts_b, hi_parts, ho_parts, b0, W1):
    return pl.pallas_call(
        _mid_body,
        grid=_GRID,
        in_specs=[pl.BlockSpec((NC, _R, F), lambda i: (0, i, 0)),
                  pl.BlockSpec((NC, _R, F), lambda i: (0, i, 0)),
                  _hist_spec(),
                  _hist_spec(),
                  pl.BlockSpec((N_HIDDEN,), lambda i: (0,)),
                  pl.BlockSpec((N_HIDDEN, OUT_FEATS), lambda i: (0, 0))],
        out_specs=pl.BlockSpec((_R, OUT_FEATS), lambda i: (i, 0)),
        out_shape=jax.ShapeDtypeStruct((N_NODES, OUT_FEATS), jnp.float32),
    )(parts_a, parts_b, hi_parts, ho_parts, b0, W1)


def _final_body(parts_ref, hi_ref, b1_ref, out_ref):
    r_in = _rsqrt_deg(hi_ref)
    t = ((parts_ref[0] + parts_ref[1]) * r_in[:, None]
         + b1_ref[...][None, :])
    m = jnp.max(t, axis=1, keepdims=True)
    e = jnp.exp(t - m)
    out_ref[...] = (t - m) - jnp.log(jnp.sum(e, axis=1, keepdims=True))


def _final_stage(parts2, hi_parts, b1):
    return pl.pallas_call(
        _final_body,
        grid=_GRID,
        in_specs=[pl.BlockSpec((NC, _R, OUT_FEATS), lambda i: (0, i, 0)),
                  _hist_spec(),
                  pl.BlockSpec((OUT_FEATS,), lambda i: (0,))],
        out_specs=pl.BlockSpec((_R, OUT_FEATS), lambda i: (i, 0)),
        out_shape=jax.ShapeDtypeStruct((N_NODES, OUT_FEATS), jnp.float32),
    )(parts2, hi_parts, b1)


def kernel(input_feature, edge_index, W0, b0, W1, b1):
    src = edge_index[0]
    dst = edge_index[1]
    srcs = src.reshape(NW, NCHUNK, K)
    dsts = dst.reshape(NW, NCHUNK, K)

    ho_parts, hi_parts = _degrees(src, dst)

    hs0_a, hs0_b = _first_stage(input_feature, W0, ho_parts)
    parts_a, parts_b = _spmm_layer1(hs0_a, hs0_b, srcs, dsts)
    hs1 = _mid_stage(parts_a, parts_b, hi_parts, ho_parts, b0, W1)
    parts2 = _spmm_layer2(hs1, srcs, dsts)
    return _final_stage(parts2, hi_parts, b1)


# speedup vs baseline: 1.0647x; 1.0415x over previous
import functools

import jax
import jax.numpy as jnp
from jax import lax
from jax.experimental import pallas as pl
from jax.experimental.pallas import tpu as pltpu
from jax.experimental.pallas import tpu_sc as plsc

N_NODES = 10000
N_EDGES = 320000
IN_FEATS = 128
N_HIDDEN = 128
OUT_FEATS = 64
F = 64

NC = 2
NS = 16
NW = NC * NS
EW = N_EDGES // NW
K = 125
NCHUNK = EW // K
NBUF = 8
G = 5
N_PAD = 10240
ROWS_PER_SUB = N_PAD // NS
ZROWS = 64

_MESH = dict(core_axis_name="c", subcore_axis_name="s", num_cores=NC,
             num_subcores=NS)
_SC_PARAMS = pltpu.CompilerParams(needs_layout_passes=False,
                                  use_tc_tiling_on_sc=False)


def _degrees(src, dst):
    mesh = plsc.VectorSubcoreMesh(**_MESH)

    @functools.partial(
        pl.kernel,
        out_type=[jax.ShapeDtypeStruct((NW, N_PAD), jnp.float32),
                  jax.ShapeDtypeStruct((NW, N_PAD), jnp.float32)],
        mesh=mesh,
        scratch_types=[pltpu.VMEM((EW,), jnp.int32),
                       pltpu.VMEM((N_PAD,), jnp.float32)],
        compiler_params=_SC_PARAMS,
    )
    def k(src_hbm, dst_hbm, out_o, out_i, idx_v, hist_v):
        c = lax.axis_index("c")
        s = lax.axis_index("s")
        wid = s * NC + c
        base = wid * EW
        zeros16 = jnp.zeros((16,), jnp.float32)
        ones16 = jnp.full((16,), 1.0, jnp.float32)

        for in_hbm, out_hbm in ((src_hbm, out_o), (dst_hbm, out_i)):
            def zero_body(i, _):
                hist_v[pl.ds(i * 16, 16)] = zeros16
                return 0

            lax.fori_loop(0, N_PAD // 16, zero_body, 0)
            pltpu.sync_copy(in_hbm.at[pl.ds(base, EW)], idx_v)

            def add_body(i, _):
                idx = idx_v[pl.ds(i * 16, 16)]
                plsc.addupdate_scatter(hist_v, [idx], ones16)
                return 0

            lax.fori_loop(0, EW // 16, add_body, 0)
            pltpu.sync_copy(hist_v, out_hbm.at[wid])

    return k(src, dst)


_SPMM_SCRATCH = [pltpu.VMEM((NCHUNK, K), jnp.int32),
                 pltpu.VMEM((NCHUNK, K), jnp.int32),
                 [pltpu.VMEM((K, F), jnp.float32)] * NBUF,
                 pltpu.VMEM((ZROWS, F), jnp.float32),
                 pltpu.VMEM_SHARED((N_PAD, F), jnp.float32),
                 [pltpu.SemaphoreType.DMA] * NBUF,
                 [pltpu.SemaphoreType.DMA] * NBUF,
                 pltpu.SemaphoreType.DMA,
                 pltpu.SemaphoreType.DMA]


def _fill_zbuf(zbuf):
    zeros16 = jnp.zeros((16,), jnp.float32)

    def zrow(r, _):
        def zcol(j, _):
            zbuf[r, pl.ds(j * 16, 16)] = zeros16
            return 0
        lax.fori_loop(0, F // 16, zcol, 0)
        return 0

    lax.fori_loop(0, ZROWS, zrow, 0)


def _zero_acc_slice(zbuf, acc, s, zsem):
    for t in range(ROWS_PER_SUB // ZROWS):
        pltpu.async_copy(
            zbuf, acc.at[pl.ds(s * ROWS_PER_SUB + t * ZROWS, ZROWS)], zsem)
    for t in range(ROWS_PER_SUB // ZROWS):
        pltpu.make_async_copy(
            zbuf, acc.at[pl.ds(s * ROWS_PER_SUB + t * ZROWS, ZROWS)],
            zsem).wait()


def _dump_acc(acc, out_hbm, c, s):
    pltpu.sync_copy(acc.at[pl.ds(s * ROWS_PER_SUB, ROWS_PER_SUB)],
                    out_hbm.at[c, pl.ds(s * ROWS_PER_SUB, ROWS_PER_SUB)])


def _fire_first_gathers(hs_hbm, src_v, rows, gsem):
    for j in range(G):
        pltpu.async_copy(hs_hbm.at[src_v.at[j]], rows[j], gsem[j])


def _ring(hs_hbm, src_v, dst_v, rows, acc, gsem, ssem):
    def fire_gather(j, b):
        pltpu.async_copy(hs_hbm.at[src_v.at[j]], rows[b], gsem[b])

    def wait_gather(j, b):
        pltpu.make_async_copy(hs_hbm.at[src_v.at[j]], rows[b],
                              gsem[b]).wait()

    def fire_scatter(j, b):
        pltpu.async_copy(rows[b], acc.at[dst_v.at[j]], ssem[b], add=True)

    def wait_scatter(j, b):
        pltpu.make_async_copy(rows[b], acc.at[dst_v.at[j]], ssem[b]).wait()

    def step(j, b, *, do_wait_s=True, do_fire_g=True):
        bg = (b + G) % NBUF
        if do_wait_s:
            wait_scatter(j + G - NBUF, bg)
        if do_fire_g:
            fire_gather(j + G, bg)
        wait_gather(j, b)
        fire_scatter(j, b)

    for b in range(NBUF):
        step(b, b, do_wait_s=(b + G - NBUF >= 0))

    def body(i, _):
        j0 = NBUF * i
        for b in range(NBUF):
            step(j0 + b, b)
        return 0

    lax.fori_loop(1, NCHUNK // NBUF - 1, body, 0)

    jl = NCHUNK - NBUF
    for b in range(NBUF):
        step(jl + b, b, do_fire_g=(jl + b + G < NCHUNK))
    for j in range(NCHUNK - (NBUF - G), NCHUNK):
        wait_scatter(j, j % NBUF)


_SPMM_KERNELS = {}


def _spmm_layer1(hs_a, hs_b, srcs, dsts):
    if 1 not in _SPMM_KERNELS:
        mesh = plsc.VectorSubcoreMesh(**_MESH)

        @functools.partial(
            pl.kernel,
            out_type=[jax.ShapeDtypeStruct((NC, N_PAD, F), jnp.float32),
                      jax.ShapeDtypeStruct((NC, N_PAD, F), jnp.float32)],
            mesh=mesh,
            scratch_types=_SPMM_SCRATCH,
            compiler_params=_SC_PARAMS,
        )
        def k(hsa_hbm, hsb_hbm, src_hbm, dst_hbm, outa_hbm, outb_hbm,
              src_v, dst_v, rows, zbuf, acc, gsem, ssem, isem_s, isem_d):
            c = lax.axis_index("c")
            s = lax.axis_index("s")
            wid = s * NC + c

            pltpu.async_copy(src_hbm.at[wid], src_v, isem_s)
            pltpu.async_copy(dst_hbm.at[wid], dst_v, isem_d)
            _fill_zbuf(zbuf)
            pltpu.make_async_copy(src_hbm.at[wid], src_v, isem_s).wait()
            pltpu.make_async_copy(dst_hbm.at[wid], dst_v, isem_d).wait()

            _fire_first_gathers(hsa_hbm, src_v, rows, gsem)
            _zero_acc_slice(zbuf, acc, s, isem_s)
            plsc.subcore_barrier()
            _ring(hsa_hbm, src_v, dst_v, rows, acc, gsem, ssem)
            plsc.subcore_barrier()
            _dump_acc(acc, outa_hbm, c, s)

            _fire_first_gathers(hsb_hbm, src_v, rows, gsem)
            _zero_acc_slice(zbuf, acc, s, isem_s)
            plsc.subcore_barrier()
            _ring(hsb_hbm, src_v, dst_v, rows, acc, gsem, ssem)
            plsc.subcore_barrier()
            _dump_acc(acc, outb_hbm, c, s)

        _SPMM_KERNELS[1] = k
    return _SPMM_KERNELS[1](hs_a, hs_b, srcs, dsts)


def _spmm_layer2(hs, srcs, dsts):
    if 2 not in _SPMM_KERNELS:
        mesh = plsc.VectorSubcoreMesh(**_MESH)

        @functools.partial(
            pl.kernel,
            out_type=jax.ShapeDtypeStruct((NC, N_PAD, F), jnp.float32),
            mesh=mesh,
            scratch_types=_SPMM_SCRATCH,
            compiler_params=_SC_PARAMS,
        )
        def k(hs_hbm, src_hbm, dst_hbm, out_hbm,
              src_v, dst_v, rows, zbuf, acc, gsem, ssem, isem_s, isem_d):
            c = lax.axis_index("c")
            s = lax.axis_index("s")
            wid = s * NC + c

            pltpu.async_copy(src_hbm.at[wid], src_v, isem_s)
            pltpu.async_copy(dst_hbm.at[wid], dst_v, isem_d)
            _fill_zbuf(zbuf)
            pltpu.make_async_copy(src_hbm.at[wid], src_v, isem_s).wait()
            pltpu.make_async_copy(dst_hbm.at[wid], dst_v, isem_d).wait()

            _fire_first_gathers(hs_hbm, src_v, rows, gsem)
            _zero_acc_slice(zbuf, acc, s, isem_s)
            plsc.subcore_barrier()
            _ring(hs_hbm, src_v, dst_v, rows, acc, gsem, ssem)
            plsc.subcore_barrier()
            _dump_acc(acc, out_hbm, c, s)

        _SPMM_KERNELS[2] = k
    return _SPMM_KERNELS[2](hs, srcs, dsts)


_R = 1024
_GRID = (N_PAD // _R,)


def _rsqrt_deg(parts_ref):
    i = pl.program_id(0)
    deg = jnp.sum(parts_ref[:, pl.ds(i * _R, _R)], axis=0)
    return lax.rsqrt(jnp.maximum(deg, 1.0))


def _hist_spec():
    return pl.BlockSpec((NW, N_PAD), lambda i: (0, 0))


def _mm_scale_body(x_ref, w_ref, ho_ref, out_a_ref, out_b_ref):
    r_out = _rsqrt_deg(ho_ref)
    h = jnp.dot(x_ref[...], w_ref[...], preferred_element_type=jnp.float32)
    hs = h * r_out[:, None]
    out_a_ref[...] = hs[:, :F]
    out_b_ref[...] = hs[:, F:]


def _first_stage(x, W0, ho_parts):
    return pl.pallas_call(
        _mm_scale_body,
        grid=_GRID,
        in_specs=[pl.BlockSpec((_R, IN_FEATS), lambda i: (i, 0)),
                  pl.BlockSpec((IN_FEATS, N_HIDDEN), lambda i: (0, 0)),
                  _hist_spec()],
        out_specs=[pl.BlockSpec((_R, F), lambda i: (i, 0)),
                   pl.BlockSpec((_R, F), lambda i: (i, 0))],
        out_shape=[jax.ShapeDtypeStruct((N_NODES, F), jnp.float32),
                   jax.ShapeDtypeStruct((N_NODES, F), jnp.float32)],
    )(x, W0, ho_parts)


def _mid_body(pa_ref, pb_ref, hi_ref, ho_ref, b0_ref, w1_ref, out_ref):
    r_in = _rsqrt_deg(hi_ref)
    r_out = _rsqrt_deg(ho_ref)
    ssum = jnp.concatenate([pa_ref[0] + pa_ref[1], pb_ref[0] + pb_ref[1]],
                           axis=1)
    h1 = jnp.maximum(ssum * r_in[:, None] + b0_ref[...][None, :], 0.0)
    h = jnp.dot(h1, w1_ref[...], preferred_element_type=jnp.float32)
    out_ref[...] = h * r_out[:, None]


def _mid_stage(parts_a, parts_b, hi_parts, ho_parts, b0, W1):
    return pl.pallas_call(
        _mid_body,
        grid=_GRID,
        in_specs=[pl.BlockSpec((NC, _R, F), lambda i: (0, i, 0)),
                  pl.BlockSpec((NC, _R, F), lambda i: (0, i, 0)),
                  _hist_spec(),
                  _hist_spec(),
                  pl.BlockSpec((N_HIDDEN,), lambda i: (0,)),
                  pl.BlockSpec((N_HIDDEN, OUT_FEATS), lambda i: (0, 0))],
        out_specs=pl.BlockSpec((_R, OUT_FEATS), lambda i: (i, 0)),
        out_shape=jax.ShapeDtypeStruct((N_NODES, OUT_FEATS), jnp.float32),
    )(parts_a, parts_b, hi_parts, ho_parts, b0, W1)


def _final_body(parts_ref, hi_ref, b1_ref, out_ref):
    r_in = _rsqrt_deg(hi_ref)
    t = ((parts_ref[0] + parts_ref[1]) * r_in[:, None]
         + b1_ref[...][None, :])
    m = jnp.max(t, axis=1, keepdims=True)
    e = jnp.exp(t - m)
    out_ref[...] = (t - m) - jnp.log(jnp.sum(e, axis=1, keepdims=True))


def _final_stage(parts2, hi_parts, b1):
    return pl.pallas_call(
        _final_body,
        grid=_GRID,
        in_specs=[pl.BlockSpec((NC, _R, OUT_FEATS), lambda i: (0, i, 0)),
                  _hist_spec(),
                  pl.BlockSpec((OUT_FEATS,), lambda i: (0,))],
        out_specs=pl.BlockSpec((_R, OUT_FEATS), lambda i: (i, 0)),
        out_shape=jax.ShapeDtypeStruct((N_NODES, OUT_FEATS), jnp.float32),
    )(parts2, hi_parts, b1)


def kernel(input_feature, edge_index, W0, b0, W1, b1):
    src = edge_index[0]
    dst = edge_index[1]
    srcs = src.reshape(NW, NCHUNK, K)
    dsts = dst.reshape(NW, NCHUNK, K)

    ho_parts, hi_parts = _degrees(src, dst)

    hs0_a, hs0_b = _first_stage(input_feature, W0, ho_parts)
    parts_a, parts_b = _spmm_layer1(hs0_a, hs0_b, srcs, dsts)
    hs1 = _mid_stage(parts_a, parts_b, hi_parts, ho_parts, b0, W1)
    parts2 = _spmm_layer2(hs1, srcs, dsts)
    return _final_stage(parts2, hi_parts, b1)


# speedup vs baseline: 1.1068x; 1.0395x over previous
import functools

import jax
import jax.numpy as jnp
from jax import lax
from jax.experimental import pallas as pl
from jax.experimental.pallas import tpu as pltpu
from jax.experimental.pallas import tpu_sc as plsc

N_NODES = 10000
N_EDGES = 320000
IN_FEATS = 128
N_HIDDEN = 128
OUT_FEATS = 64
F = 64

NC = 2
NS = 16
NW = NC * NS
EW = N_EDGES // NW
K = 125
NCHUNK = EW // K
NBUF = 8
G = 5
N_PAD = 10240
ROWS_PER_SUB = N_PAD // NS
ZROWS = 64

_MESH = dict(core_axis_name="c", subcore_axis_name="s", num_cores=NC,
             num_subcores=NS)
_SC_PARAMS = pltpu.CompilerParams(needs_layout_passes=False,
                                  use_tc_tiling_on_sc=False)


def _degrees(edge_index):
    mesh = plsc.VectorSubcoreMesh(**_MESH)

    @functools.partial(
        pl.kernel,
        out_type=[jax.ShapeDtypeStruct((NW, N_PAD), jnp.float32),
                  jax.ShapeDtypeStruct((NW, N_PAD), jnp.float32)],
        mesh=mesh,
        scratch_types=[pltpu.VMEM((EW,), jnp.int32),
                       pltpu.VMEM((N_PAD,), jnp.float32)],
        compiler_params=_SC_PARAMS,
    )
    def k(e_hbm, out_o, out_i, idx_v, hist_v):
        c = lax.axis_index("c")
        s = lax.axis_index("s")
        wid = s * NC + c
        base = wid * EW
        zeros16 = jnp.zeros((16,), jnp.float32)
        ones16 = jnp.full((16,), 1.0, jnp.float32)

        for d, out_hbm in ((0, out_o), (1, out_i)):
            def zero_body(i, _):
                hist_v[pl.ds(i * 16, 16)] = zeros16
                return 0

            lax.fori_loop(0, N_PAD // 16, zero_body, 0)
            pltpu.sync_copy(e_hbm.at[d, pl.ds(base, EW)], idx_v)

            def add_body(i, _):
                idx = idx_v[pl.ds(i * 16, 16)]
                plsc.addupdate_scatter(hist_v, [idx], ones16)
                return 0

            lax.fori_loop(0, EW // 16, add_body, 0)
            pltpu.sync_copy(hist_v, out_hbm.at[wid])

    return k(edge_index)


_SPMM_SCRATCH = [pltpu.VMEM((NCHUNK, K), jnp.int32),
                 pltpu.VMEM((NCHUNK, K), jnp.int32),
                 [pltpu.VMEM((K, F), jnp.float32)] * NBUF,
                 pltpu.VMEM((ZROWS, F), jnp.float32),
                 pltpu.VMEM_SHARED((N_PAD, F), jnp.float32),
                 [pltpu.SemaphoreType.DMA] * NBUF,
                 [pltpu.SemaphoreType.DMA] * NBUF,
                 pltpu.SemaphoreType.DMA,
                 pltpu.SemaphoreType.DMA]


def _fill_zbuf(zbuf):
    zeros16 = jnp.zeros((16,), jnp.float32)

    def zrow(r, _):
        def zcol(j, _):
            zbuf[r, pl.ds(j * 16, 16)] = zeros16
            return 0
        lax.fori_loop(0, F // 16, zcol, 0)
        return 0

    lax.fori_loop(0, ZROWS, zrow, 0)


def _zero_acc_slice(zbuf, acc, s, zsem):
    for t in range(ROWS_PER_SUB // ZROWS):
        pltpu.async_copy(
            zbuf, acc.at[pl.ds(s * ROWS_PER_SUB + t * ZROWS, ZROWS)], zsem)
    for t in range(ROWS_PER_SUB // ZROWS):
        pltpu.make_async_copy(
            zbuf, acc.at[pl.ds(s * ROWS_PER_SUB + t * ZROWS, ZROWS)],
            zsem).wait()


def _dump_acc(acc, out_hbm, c, s):
    pltpu.sync_copy(acc.at[pl.ds(s * ROWS_PER_SUB, ROWS_PER_SUB)],
                    out_hbm.at[c, pl.ds(s * ROWS_PER_SUB, ROWS_PER_SUB)])


def _fire_first_gathers(hs_hbm, src_v, rows, gsem):
    for j in range(G):
        pltpu.async_copy(hs_hbm.at[src_v.at[j]], rows[j], gsem[j])


def _ring(hs_hbm, src_v, dst_v, rows, acc, gsem, ssem):
    def fire_gather(j, b):
        pltpu.async_copy(hs_hbm.at[src_v.at[j]], rows[b], gsem[b])

    def wait_gather(j, b):
        pltpu.make_async_copy(hs_hbm.at[src_v.at[j]], rows[b],
                              gsem[b]).wait()

    def fire_scatter(j, b):
        pltpu.async_copy(rows[b], acc.at[dst_v.at[j]], ssem[b], add=True)

    def wait_scatter(j, b):
        pltpu.make_async_copy(rows[b], acc.at[dst_v.at[j]], ssem[b]).wait()

    def step(j, b, *, do_wait_s=True, do_fire_g=True):
        bg = (b + G) % NBUF
        if do_wait_s:
            wait_scatter(j + G - NBUF, bg)
        if do_fire_g:
            fire_gather(j + G, bg)
        wait_gather(j, b)
        fire_scatter(j, b)

    for b in range(NBUF):
        step(b, b, do_wait_s=(b + G - NBUF >= 0))

    def body(i, _):
        j0 = NBUF * i
        for b in range(NBUF):
            step(j0 + b, b)
        return 0

    lax.fori_loop(1, NCHUNK // NBUF - 1, body, 0)

    jl = NCHUNK - NBUF
    for b in range(NBUF):
        step(jl + b, b, do_fire_g=(jl + b + G < NCHUNK))
    for j in range(NCHUNK - (NBUF - G), NCHUNK):
        wait_scatter(j, j % NBUF)


_SPMM_KERNELS = {}


def _spmm_layer1(hs_a, hs_b, ei4):
    if 1 not in _SPMM_KERNELS:
        mesh = plsc.VectorSubcoreMesh(**_MESH)

        @functools.partial(
            pl.kernel,
            out_type=[jax.ShapeDtypeStruct((NC, N_PAD, F), jnp.float32),
                      jax.ShapeDtypeStruct((NC, N_PAD, F), jnp.float32)],
            mesh=mesh,
            scratch_types=_SPMM_SCRATCH,
            compiler_params=_SC_PARAMS,
        )
        def k(hsa_hbm, hsb_hbm, ei_hbm, outa_hbm, outb_hbm,
              src_v, dst_v, rows, zbuf, acc, gsem, ssem, isem_s, isem_d):
            c = lax.axis_index("c")
            s = lax.axis_index("s")
            wid = s * NC + c

            pltpu.async_copy(ei_hbm.at[0, wid], src_v, isem_s)
            pltpu.async_copy(ei_hbm.at[1, wid], dst_v, isem_d)
            _fill_zbuf(zbuf)
            pltpu.make_async_copy(ei_hbm.at[0, wid], src_v, isem_s).wait()
            pltpu.make_async_copy(ei_hbm.at[1, wid], dst_v, isem_d).wait()

            _fire_first_gathers(hsa_hbm, src_v, rows, gsem)
            _zero_acc_slice(zbuf, acc, s, isem_s)
            plsc.subcore_barrier()
            _ring(hsa_hbm, src_v, dst_v, rows, acc, gsem, ssem)
            plsc.subcore_barrier()
            _dump_acc(acc, outa_hbm, c, s)

            _fire_first_gathers(hsb_hbm, src_v, rows, gsem)
            _zero_acc_slice(zbuf, acc, s, isem_s)
            plsc.subcore_barrier()
            _ring(hsb_hbm, src_v, dst_v, rows, acc, gsem, ssem)
            plsc.subcore_barrier()
            _dump_acc(acc, outb_hbm, c, s)

        _SPMM_KERNELS[1] = k
    return _SPMM_KERNELS[1](hs_a, hs_b, ei4)


def _spmm_layer2(hs, ei4):
    if 2 not in _SPMM_KERNELS:
        mesh = plsc.VectorSubcoreMesh(**_MESH)

        @functools.partial(
            pl.kernel,
            out_type=jax.ShapeDtypeStruct((NC, N_PAD, F), jnp.float32),
            mesh=mesh,
            scratch_types=_SPMM_SCRATCH,
            compiler_params=_SC_PARAMS,
        )
        def k(hs_hbm, ei_hbm, out_hbm,
              src_v, dst_v, rows, zbuf, acc, gsem, ssem, isem_s, isem_d):
            c = lax.axis_index("c")
            s = lax.axis_index("s")
            wid = s * NC + c

            pltpu.async_copy(ei_hbm.at[0, wid], src_v, isem_s)
            pltpu.async_copy(ei_hbm.at[1, wid], dst_v, isem_d)
            _fill_zbuf(zbuf)
            pltpu.make_async_copy(ei_hbm.at[0, wid], src_v, isem_s).wait()
            pltpu.make_async_copy(ei_hbm.at[1, wid], dst_v, isem_d).wait()

            _fire_first_gathers(hs_hbm, src_v, rows, gsem)
            _zero_acc_slice(zbuf, acc, s, isem_s)
            plsc.subcore_barrier()
            _ring(hs_hbm, src_v, dst_v, rows, acc, gsem, ssem)
            plsc.subcore_barrier()
            _dump_acc(acc, out_hbm, c, s)

        _SPMM_KERNELS[2] = k
    return _SPMM_KERNELS[2](hs, ei4)


_R = 1024
_GRID = (N_PAD // _R,)


def _rsqrt_deg(parts_ref):
    i = pl.program_id(0)
    deg = jnp.sum(parts_ref[:, pl.ds(i * _R, _R)], axis=0)
    return lax.rsqrt(jnp.maximum(deg, 1.0))


def _hist_spec():
    return pl.BlockSpec((NW, N_PAD), lambda i: (0, 0))


def _mm_scale_body(x_ref, w_ref, ho_ref, out_a_ref, out_b_ref):
    r_out = _rsqrt_deg(ho_ref)
    h = jnp.dot(x_ref[...], w_ref[...], preferred_element_type=jnp.float32)
    hs = h * r_out[:, None]
    out_a_ref[...] = hs[:, :F]
    out_b_ref[...] = hs[:, F:]


def _first_stage(x, W0, ho_parts):
    return pl.pallas_call(
        _mm_scale_body,
        grid=_GRID,
        in_specs=[pl.BlockSpec((_R, IN_FEATS), lambda i: (i, 0)),
                  pl.BlockSpec((IN_FEATS, N_HIDDEN), lambda i: (0, 0)),
                  _hist_spec()],
        out_specs=[pl.BlockSpec((_R, F), lambda i: (i, 0)),
                   pl.BlockSpec((_R, F), lambda i: (i, 0))],
        out_shape=[jax.ShapeDtypeStruct((N_NODES, F), jnp.float32),
                   jax.ShapeDtypeStruct((N_NODES, F), jnp.float32)],
    )(x, W0, ho_parts)


def _mid_body(pa_ref, pb_ref, hi_ref, ho_ref, b0_ref, w1_ref, out_ref):
    r_in = _rsqrt_deg(hi_ref)
    r_out = _rsqrt_deg(ho_ref)
    ssum = jnp.concatenate([pa_ref[0] + pa_ref[1], pb_ref[0] + pb_ref[1]],
                           axis=1)
    h1 = jnp.maximum(ssum * r_in[:, None] + b0_ref[...][None, :], 0.0)
    h = jnp.dot(h1, w1_ref[...], preferred_element_type=jnp.float32)
    out_ref[...] = h * r_out[:, None]


def _mid_stage(parts_a, parts_b, hi_parts, ho_parts, b0, W1):
    return pl.pallas_call(
        _mid_body,
        grid=_GRID,
        in_specs=[pl.BlockSpec((NC, _R, F), lambda i: (0, i, 0)),
                  pl.BlockSpec((NC, _R, F), lambda i: (0, i, 0)),
                  _hist_spec(),
                  _hist_spec(),
                  pl.BlockSpec((N_HIDDEN,), lambda i: (0,)),
                  pl.BlockSpec((N_HIDDEN, OUT_FEATS), lambda i: (0, 0))],
        out_specs=pl.BlockSpec((_R, OUT_FEATS), lambda i: (i, 0)),
        out_shape=jax.ShapeDtypeStruct((N_NODES, OUT_FEATS), jnp.float32),
    )(parts_a, parts_b, hi_parts, ho_parts, b0, W1)


def _final_body(parts_ref, hi_ref, b1_ref, out_ref):
    r_in = _rsqrt_deg(hi_ref)
    t = ((parts_ref[0] + parts_ref[1]) * r_in[:, None]
         + b1_ref[...][None, :])
    m = jnp.max(t, axis=1, keepdims=True)
    e = jnp.exp(t - m)
    out_ref[...] = (t - m) - jnp.log(jnp.sum(e, axis=1, keepdims=True))


def _final_stage(parts2, hi_parts, b1):
    return pl.pallas_call(
        _final_body,
        grid=_GRID,
        in_specs=[pl.BlockSpec((NC, _R, OUT_FEATS), lambda i: (0, i, 0)),
                  _hist_spec(),
                  pl.BlockSpec((OUT_FEATS,), lambda i: (0,))],
        out_specs=pl.BlockSpec((_R, OUT_FEATS), lambda i: (i, 0)),
        out_shape=jax.ShapeDtypeStruct((N_NODES, OUT_FEATS), jnp.float32),
    )(parts2, hi_parts, b1)


def kernel(input_feature, edge_index, W0, b0, W1, b1):
    ei4 = edge_index.reshape(2, NW, NCHUNK, K)

    ho_parts, hi_parts = _degrees(edge_index)

    hs0_a, hs0_b = _first_stage(input_feature, W0, ho_parts)
    parts_a, parts_b = _spmm_layer1(hs0_a, hs0_b, ei4)
    hs1 = _mid_stage(parts_a, parts_b, hi_parts, ho_parts, b0, W1)
    parts2 = _spmm_layer2(hs1, ei4)
    return _final_stage(parts2, hi_parts, b1)
